# Initial kernel scaffold; baseline (speedup 1.0000x reference)
import jax
import jax.numpy as jnp
from jax.experimental import pallas as pl


def kernel(bins, weights, n_samples, det):
    raise NotImplementedError("write your pallas kernel here")


# speedup vs baseline: 18.6203x; 18.6203x over previous
import functools

import jax
import jax.numpy as jnp
from jax import lax
from jax.experimental import pallas as pl
from jax.experimental.pallas import tpu as pltpu
from jax.experimental.pallas import tpu_sc as plsc

_NC = 2
_NS = 16
_L = 16
_NW = _NC * _NS


def _sample_pdf_sc(bins, weights, u, u_shared):
    N, M = bins.shape
    S = u.shape[-1]
    assert M % _L == 0 and S % _L == 0
    assert N % _NW == 0
    rays_per_w = N // _NW
    CH = 64 if rays_per_w % 64 == 0 else rays_per_w
    stages = rays_per_w // CH
    MC = M // _L
    QC = S // _L
    n_search = (M - 1).bit_length()

    mesh = plsc.VectorSubcoreMesh(core_axis_name="c", subcore_axis_name="s")
    scratch = [
        pltpu.VMEM((CH, M), jnp.float32),
        pltpu.VMEM((CH, M - 1), jnp.float32),
        pltpu.VMEM((CH, M), jnp.float32),
        pltpu.VMEM((CH, S), jnp.float32),
        pltpu.VMEM((S,) if u_shared else (CH, S), jnp.float32),
    ]

    @functools.partial(
        pl.kernel,
        out_type=jax.ShapeDtypeStruct((N, S), jnp.float32),
        mesh=mesh,
        scratch_types=scratch,
        compiler_params=pltpu.CompilerParams(needs_layout_passes=False),
    )
    def run(bins_hbm, w_hbm, u_hbm, out_hbm, binsv, wv, crawv, outv, uv):
        wid = lax.axis_index("s") * _NC + lax.axis_index("c")
        iota = lax.broadcasted_iota(jnp.int32, (_L,), 0)
        if u_shared:
            pltpu.sync_copy(u_hbm, uv)

        def stage_body(s, _):
            base = wid * rays_per_w + s * CH
            pltpu.sync_copy(bins_hbm.at[pl.ds(base, CH)], binsv)
            pltpu.sync_copy(w_hbm.at[pl.ds(base, CH)], wv)
            if not u_shared:
                pltpu.sync_copy(u_hbm.at[pl.ds(base, CH)], uv)

            def ray_body(r, _):
                rfull = jnp.full((_L,), r, jnp.int32)
                carry = jnp.float32(0.0)
                for c in range(MC):
                    if c < MC - 1:
                        wc = wv[r, pl.ds(c * _L, _L)] + jnp.float32(1e-5)
                    else:
                        idx = jnp.minimum(c * _L + iota, M - 2)
                        wc = plsc.load_gather(wv, [rfull, idx]) + jnp.float32(1e-5)
                        wc = jnp.where(iota < _L - 1, wc, jnp.float32(0.0))
                    incl = plsc.cumsum(wc)
                    crawv[r, pl.ds(c * _L, _L)] = incl - wc + carry
                    carry = carry + jnp.sum(wc)
                total = carry
                thresh = jnp.float32(1e-5) * total

                for q in range(QC):
                    if u_shared:
                        uq = uv[pl.ds(q * _L, _L)]
                    else:
                        uq = uv[r, pl.ds(q * _L, _L)]
                    target = uq * total
                    lo = jnp.full((_L,), 1, jnp.int32)
                    hi = jnp.full((_L,), M, jnp.int32)
                    for _ in range(n_search):
                        mid = lax.shift_right_logical(lo + hi, 1)
                        cval = plsc.load_gather(crawv, [rfull, mid])
                        le = cval <= target
                        lo = jnp.where(le, mid + 1, lo)
                        hi = jnp.where(le, hi, mid)
                    below = lo - 1
                    above = jnp.minimum(lo, M - 1)
                    c0 = plsc.load_gather(crawv, [rfull, below])
                    c1 = plsc.load_gather(crawv, [rfull, above])
                    b0 = plsc.load_gather(binsv, [rfull, below])
                    b1 = plsc.load_gather(binsv, [rfull, above])
                    denom = c1 - c0
                    denomc = jnp.where(denom < thresh, total, denom)
                    t = (target - c0) / denomc
                    outv[r, pl.ds(q * _L, _L)] = b0 + t * (b1 - b0)
                return 0

            lax.fori_loop(0, CH, ray_body, 0)
            pltpu.sync_copy(outv, out_hbm.at[pl.ds(base, CH)])
            return 0

        lax.fori_loop(0, stages, stage_body, 0)

    return run(bins, weights, u)


def kernel(bins, weights, n_samples, det):
    N, _ = bins.shape
    S = bins.shape[-1]

    def det_path():
        u = jnp.linspace(0.5 / n_samples, 1.0 - 0.5 / n_samples, S,
                         dtype=jnp.float32)
        return _sample_pdf_sc(bins, weights, u, True)

    def rand_path():
        u = jax.random.uniform(jax.random.key(1), (N, S), dtype=jnp.float32)
        return _sample_pdf_sc(bins, weights, u, False)

    return lax.cond(jnp.asarray(det) != 0, det_path, rand_path)


# speedup vs baseline: 61.5129x; 3.3035x over previous
import functools

import jax
import jax.numpy as jnp
from jax import lax
from jax.experimental import pallas as pl
from jax.experimental.pallas import tpu as pltpu
from jax.experimental.pallas import tpu_sc as plsc

_NC = 2
_NS = 16
_L = 16
_NW = _NC * _NS


def _sample_pdf_sc(bins, weights, u, u_shared):
    N, M = bins.shape
    S = u.shape[-1]
    assert M % _L == 0 and S % _L == 0
    assert N % _NW == 0
    rays_per_w = N // _NW
    CH = 64 if rays_per_w % 64 == 0 else rays_per_w
    stages = rays_per_w // CH
    MC = M // _L
    QC = S // _L
    n_search = (M - 1).bit_length()

    mesh = plsc.VectorSubcoreMesh(core_axis_name="c", subcore_axis_name="s")
    scratch = [
        pltpu.VMEM((CH, M), jnp.float32),
        pltpu.VMEM((CH, M - 1), jnp.float32),
        pltpu.VMEM((CH, M), jnp.float32),
        pltpu.VMEM((CH, S), jnp.float32),
        pltpu.VMEM((S,) if u_shared else (CH, S), jnp.float32),
    ]

    @functools.partial(
        pl.kernel,
        out_type=jax.ShapeDtypeStruct((N, S), jnp.float32),
        mesh=mesh,
        scratch_types=scratch,
        compiler_params=pltpu.CompilerParams(needs_layout_passes=False),
    )
    def run(bins_hbm, w_hbm, u_hbm, out_hbm, binsv, wv, crawv, outv, uv):
        wid = lax.axis_index("s") * _NC + lax.axis_index("c")
        iota = lax.broadcasted_iota(jnp.int32, (_L,), 0)
        if u_shared:
            pltpu.sync_copy(u_hbm, uv)

        def stage_body(s, _):
            base = wid * rays_per_w + s * CH
            pltpu.sync_copy(bins_hbm.at[pl.ds(base, CH)], binsv)
            pltpu.sync_copy(w_hbm.at[pl.ds(base, CH)], wv)
            if not u_shared:
                pltpu.sync_copy(u_hbm.at[pl.ds(base, CH)], uv)

            @plsc.parallel_loop(0, CH)
            def ray_body(r):
                rfull = jnp.full((_L,), r, jnp.int32)
                carry = jnp.float32(0.0)
                for c in range(MC):
                    if c < MC - 1:
                        wc = wv[r, pl.ds(c * _L, _L)] + jnp.float32(1e-5)
                    else:
                        idx = jnp.minimum(c * _L + iota, M - 2)
                        wc = plsc.load_gather(wv, [rfull, idx]) + jnp.float32(1e-5)
                        wc = jnp.where(iota < _L - 1, wc, jnp.float32(0.0))
                    incl = plsc.cumsum(wc)
                    crawv[r, pl.ds(c * _L, _L)] = incl - wc + carry
                    carry = carry + jnp.sum(wc)
                total = carry
                thresh = jnp.float32(1e-5) * total

                for q in range(QC):
                    if u_shared:
                        uq = uv[pl.ds(q * _L, _L)]
                    else:
                        uq = uv[r, pl.ds(q * _L, _L)]
                    target = uq * total
                    lo = jnp.full((_L,), 1, jnp.int32)
                    hi = jnp.full((_L,), M, jnp.int32)
                    for _ in range(n_search):
                        mid = lax.shift_right_logical(lo + hi, 1)
                        cval = plsc.load_gather(crawv, [rfull, mid])
                        le = cval <= target
                        lo = jnp.where(le, mid + 1, lo)
                        hi = jnp.where(le, hi, mid)
                    below = lo - 1
                    above = jnp.minimum(lo, M - 1)
                    c0 = plsc.load_gather(crawv, [rfull, below])
                    c1 = plsc.load_gather(crawv, [rfull, above])
                    b0 = plsc.load_gather(binsv, [rfull, below])
                    b1 = plsc.load_gather(binsv, [rfull, above])
                    denom = c1 - c0
                    denomc = jnp.where(denom < thresh, total, denom)
                    t = (target - c0) / denomc
                    outv[r, pl.ds(q * _L, _L)] = b0 + t * (b1 - b0)

            pltpu.sync_copy(outv, out_hbm.at[pl.ds(base, CH)])
            return 0

        lax.fori_loop(0, stages, stage_body, 0)

    return run(bins, weights, u)


def kernel(bins, weights, n_samples, det):
    N, _ = bins.shape
    S = bins.shape[-1]

    def det_path():
        u = jnp.linspace(0.5 / n_samples, 1.0 - 0.5 / n_samples, S,
                         dtype=jnp.float32)
        return _sample_pdf_sc(bins, weights, u, True)

    def rand_path():
        u = jax.random.uniform(jax.random.key(1), (N, S), dtype=jnp.float32)
        return _sample_pdf_sc(bins, weights, u, False)

    return lax.cond(jnp.asarray(det) != 0, det_path, rand_path)


# speedup vs baseline: 82.0257x; 1.3335x over previous
import functools

import jax
import jax.numpy as jnp
from jax import lax
from jax.experimental import pallas as pl
from jax.experimental.pallas import tpu as pltpu
from jax.experimental.pallas import tpu_sc as plsc

_NC = 2
_NS = 16
_L = 16
_NW = _NC * _NS


def _sample_pdf_sc(bins, weights, u, u_shared, gridp=None):
    N, M = bins.shape
    S = u.shape[-1]
    assert M % _L == 0 and S % _L == 0
    assert N % _NW == 0
    rays_per_w = N // _NW
    CH = 64 if rays_per_w % 64 == 0 else rays_per_w
    stages = rays_per_w // CH
    MC = M // _L
    QC = S // _L
    n_search = (M - 1).bit_length()
    HW = (S + 1 + _L - 1) // _L * _L

    mesh = plsc.VectorSubcoreMesh(core_axis_name="c", subcore_axis_name="s")
    scratch = [
        pltpu.VMEM((CH, M), jnp.float32),
        pltpu.VMEM((CH, M - 1), jnp.float32),
        pltpu.VMEM((CH, M), jnp.float32),
        pltpu.VMEM((CH, S), jnp.float32),
        pltpu.VMEM((S,) if u_shared else (CH, S), jnp.float32),
        pltpu.VMEM((CH, HW), jnp.int32),
        pltpu.VMEM((2 * _L,), jnp.float32),
    ]

    @functools.partial(
        pl.kernel,
        out_type=jax.ShapeDtypeStruct((N, S), jnp.float32),
        mesh=mesh,
        scratch_types=scratch,
        compiler_params=pltpu.CompilerParams(needs_layout_passes=False),
    )
    def run(bins_hbm, w_hbm, u_hbm, gridp_hbm, out_hbm,
            binsv, wv, crawv, outv, uv, histv, gridv):
        wid = lax.axis_index("s") * _NC + lax.axis_index("c")
        iota = lax.broadcasted_iota(jnp.int32, (_L,), 0)
        zeros_i = jnp.zeros((_L,), jnp.int32)
        if u_shared:
            pltpu.sync_copy(u_hbm, uv)
            pltpu.sync_copy(gridp_hbm, gridv)
            gstep = gridv[pl.ds(0, _L)]
            gbos = gridv[pl.ds(_L, _L)]

            @plsc.parallel_loop(0, CH)
            def init_hist(r):
                for h in range(HW // _L):
                    histv[r, pl.ds(h * _L, _L)] = zeros_i

        def stage_body(s, _):
            base = wid * rays_per_w + s * CH
            pltpu.sync_copy(bins_hbm.at[pl.ds(base, CH)], binsv)
            pltpu.sync_copy(w_hbm.at[pl.ds(base, CH)], wv)
            if not u_shared:
                pltpu.sync_copy(u_hbm.at[pl.ds(base, CH)], uv)

            @plsc.parallel_loop(0, CH)
            def ray_body(r):
                rfull = jnp.full((_L,), r, jnp.int32)
                carry = jnp.float32(0.0)
                for c in range(MC):
                    if c < MC - 1:
                        wc = wv[r, pl.ds(c * _L, _L)] + jnp.float32(1e-5)
                    else:
                        idx = jnp.minimum(c * _L + iota, M - 2)
                        wc = plsc.load_gather(wv, [rfull, idx]) + jnp.float32(1e-5)
                        wc = jnp.where(iota < _L - 1, wc, jnp.float32(0.0))
                    incl = plsc.cumsum(wc)
                    crawv[r, pl.ds(c * _L, _L)] = incl - wc + carry
                    carry = carry + jnp.sum(wc)
                total = carry
                thresh = jnp.float32(1e-5) * total

                if u_shared:
                    ainv = jnp.float32(1.0) / (gstep * total)
                    ones_i = jnp.ones((_L,), jnp.int32)
                    for c in range(MC):
                        cr = crawv[r, pl.ds(c * _L, _L)]
                        y = cr * ainv - gbos
                        ti = y.astype(jnp.int32)
                        adj = (y > ti.astype(jnp.float32)).astype(jnp.int32)
                        p = jnp.clip(ti + adj, 0, S)
                        plsc.addupdate_scatter(histv, [rfull, p], ones_i)

                    carry2 = jnp.int32(0)
                    for q in range(QC):
                        hv = histv[r, pl.ds(q * _L, _L)]
                        histv[r, pl.ds(q * _L, _L)] = zeros_i
                        inds = plsc.cumsum(hv) + carry2
                        carry2 = carry2 + jnp.sum(hv)
                        uq = uv[pl.ds(q * _L, _L)]
                        target = uq * total
                        below = inds - 1
                        above = jnp.minimum(inds, M - 1)
                        c0 = plsc.load_gather(crawv, [rfull, below])
                        c1 = plsc.load_gather(crawv, [rfull, above])
                        b0 = plsc.load_gather(binsv, [rfull, below])
                        b1 = plsc.load_gather(binsv, [rfull, above])
                        denom = c1 - c0
                        denomc = jnp.where(denom < thresh, total, denom)
                        t = (target - c0) / denomc
                        outv[r, pl.ds(q * _L, _L)] = b0 + t * (b1 - b0)
                else:
                    for q in range(QC):
                        uq = uv[r, pl.ds(q * _L, _L)]
                        target = uq * total
                        lo = jnp.full((_L,), 1, jnp.int32)
                        hi = jnp.full((_L,), M, jnp.int32)
                        for _ in range(n_search):
                            mid = lax.shift_right_logical(lo + hi, 1)
                            cval = plsc.load_gather(crawv, [rfull, mid])
                            le = cval <= target
                            lo = jnp.where(le, mid + 1, lo)
                            hi = jnp.where(le, hi, mid)
                        below = lo - 1
                        above = jnp.minimum(lo, M - 1)
                        c0 = plsc.load_gather(crawv, [rfull, below])
                        c1 = plsc.load_gather(crawv, [rfull, above])
                        b0 = plsc.load_gather(binsv, [rfull, below])
                        b1 = plsc.load_gather(binsv, [rfull, above])
                        denom = c1 - c0
                        denomc = jnp.where(denom < thresh, total, denom)
                        t = (target - c0) / denomc
                        outv[r, pl.ds(q * _L, _L)] = b0 + t * (b1 - b0)

            pltpu.sync_copy(outv, out_hbm.at[pl.ds(base, CH)])
            return 0

        lax.fori_loop(0, stages, stage_body, 0)

    if gridp is None:
        gridp = jnp.zeros((2 * _L,), jnp.float32)
    return run(bins, weights, u, gridp)


def kernel(bins, weights, n_samples, det):
    N, _ = bins.shape
    S = bins.shape[-1]

    def det_path():
        lo = jnp.float32(0.5) / n_samples
        hi = jnp.float32(1.0) - jnp.float32(0.5) / n_samples
        u = jnp.linspace(lo, hi, S, dtype=jnp.float32)
        step = (hi - lo) / jnp.float32(max(S - 1, 1))
        step = jnp.where(step > 0, step, jnp.float32(1e-30))
        gridp = jnp.concatenate([
            jnp.full((_L,), step, jnp.float32),
            jnp.full((_L,), lo / step, jnp.float32),
        ])
        return _sample_pdf_sc(bins, weights, u, True, gridp)

    def rand_path():
        u = jax.random.uniform(jax.random.key(1), (N, S), dtype=jnp.float32)
        return _sample_pdf_sc(bins, weights, u, False)

    return lax.cond(jnp.asarray(det) != 0, det_path, rand_path)


# speedup vs baseline: 94.9587x; 1.1577x over previous
import functools

import jax
import jax.numpy as jnp
from jax import lax
from jax.experimental import pallas as pl
from jax.experimental.pallas import tpu as pltpu
from jax.experimental.pallas import tpu_sc as plsc

_NC = 2
_NS = 16
_L = 16
_NW = _NC * _NS


def _sample_pdf_sc(bins, weights, u, u_shared, gridp=None):
    N, M = bins.shape
    S = u.shape[-1]
    assert M % _L == 0 and S % _L == 0
    assert N % _NW == 0
    rays_per_w = N // _NW
    CH = 64 if rays_per_w % 64 == 0 else rays_per_w
    stages = rays_per_w // CH
    MC = M // _L
    QC = S // _L
    n_search = (M - 1).bit_length()
    HW = (S + 1 + _L - 1) // _L * _L
    ring = u_shared and stages % 2 == 0 and stages >= 2

    if gridp is None:
        gridp = jnp.zeros((2 * _L,), jnp.float32)

    mesh = plsc.VectorSubcoreMesh(core_axis_name="c", subcore_axis_name="s")
    nbuf = 2 if ring else 1
    scratch = dict(
        binsv=[pltpu.VMEM((CH, M), jnp.float32) for _ in range(nbuf)],
        wv=[pltpu.VMEM((CH, M - 1), jnp.float32) for _ in range(nbuf)],
        outv=[pltpu.VMEM((CH, S), jnp.float32) for _ in range(nbuf)],
        crawv=pltpu.VMEM((CH, M), jnp.float32),
        uv=pltpu.VMEM((S,) if u_shared else (CH, S), jnp.float32),
        histv=pltpu.VMEM((CH, HW), jnp.int32),
        gridv=pltpu.VMEM((2 * _L,), jnp.float32),
        in_sems=[pltpu.SemaphoreType.DMA for _ in range(nbuf)],
        out_sems=[pltpu.SemaphoreType.DMA for _ in range(nbuf)],
    )

    @functools.partial(
        pl.kernel,
        out_type=jax.ShapeDtypeStruct((N, S), jnp.float32),
        mesh=mesh,
        scratch_types=scratch,
        compiler_params=pltpu.CompilerParams(needs_layout_passes=False),
    )
    def run(bins_hbm, w_hbm, u_hbm, gridp_hbm, out_hbm, *, binsv, wv, outv,
            crawv, uv, histv, gridv, in_sems, out_sems):
        wid = lax.axis_index("s") * _NC + lax.axis_index("c")
        iota = lax.broadcasted_iota(jnp.int32, (_L,), 0)
        zeros_i = jnp.zeros((_L,), jnp.int32)
        wbase = wid * rays_per_w

        if u_shared:
            pltpu.sync_copy(u_hbm, uv)
            pltpu.sync_copy(gridp_hbm, gridv)
            gstep = gridv[pl.ds(0, _L)]
            gbos = gridv[pl.ds(_L, _L)]

            @plsc.parallel_loop(0, CH)
            def init_hist(r):
                for h in range(HW // _L):
                    histv[r, pl.ds(h * _L, _L)] = zeros_i

        def compute_chunk(base, bv, wvb, ov):
            @plsc.parallel_loop(0, CH)
            def ray_body(r):
                rfull = jnp.full((_L,), r, jnp.int32)
                carry = jnp.float32(0.0)
                for c in range(MC):
                    if c < MC - 1:
                        wc = wvb[r, pl.ds(c * _L, _L)] + jnp.float32(1e-5)
                    else:
                        idx = jnp.minimum(c * _L + iota, M - 2)
                        wc = plsc.load_gather(wvb, [rfull, idx]) + jnp.float32(1e-5)
                        wc = jnp.where(iota < _L - 1, wc, jnp.float32(0.0))
                    incl = plsc.cumsum(wc)
                    crawv[r, pl.ds(c * _L, _L)] = incl - wc + carry
                    carry = carry + jnp.sum(wc)
                total = carry
                thresh = jnp.float32(1e-5) * total

                def interp(q, inds, target):
                    below = inds - 1
                    above = jnp.minimum(inds, M - 1)
                    c0 = plsc.load_gather(crawv, [rfull, below])
                    c1 = plsc.load_gather(crawv, [rfull, above])
                    b0 = plsc.load_gather(bv, [rfull, below])
                    b1 = plsc.load_gather(bv, [rfull, above])
                    denom = c1 - c0
                    denomc = jnp.where(denom < thresh, total, denom)
                    t = (target - c0) / denomc
                    ov[r, pl.ds(q * _L, _L)] = b0 + t * (b1 - b0)

                if u_shared:
                    ainv = jnp.float32(1.0) / (gstep * total)
                    ones_i = jnp.ones((_L,), jnp.int32)
                    for c in range(MC):
                        cr = crawv[r, pl.ds(c * _L, _L)]
                        y = cr * ainv - gbos
                        ti = y.astype(jnp.int32)
                        adj = (y > ti.astype(jnp.float32)).astype(jnp.int32)
                        p = jnp.clip(ti + adj, 0, S)
                        plsc.addupdate_scatter(histv, [rfull, p], ones_i)

                    carry2 = jnp.int32(0)
                    for q in range(QC):
                        hv = histv[r, pl.ds(q * _L, _L)]
                        histv[r, pl.ds(q * _L, _L)] = zeros_i
                        inds = plsc.cumsum(hv) + carry2
                        carry2 = carry2 + jnp.sum(hv)
                        uq = uv[pl.ds(q * _L, _L)]
                        interp(q, inds, uq * total)
                else:
                    for q in range(QC):
                        uq = uv[r, pl.ds(q * _L, _L)]
                        target = uq * total
                        lo = jnp.full((_L,), 1, jnp.int32)
                        hi = jnp.full((_L,), M, jnp.int32)
                        for _ in range(n_search):
                            mid = lax.shift_right_logical(lo + hi, 1)
                            cval = plsc.load_gather(crawv, [rfull, mid])
                            le = cval <= target
                            lo = jnp.where(le, mid + 1, lo)
                            hi = jnp.where(le, hi, mid)
                        interp(q, lo, target)

        if not ring:
            def stage_body(s, _):
                base = wbase + s * CH
                pltpu.sync_copy(bins_hbm.at[pl.ds(base, CH)], binsv[0])
                pltpu.sync_copy(w_hbm.at[pl.ds(base, CH)], wv[0])
                if not u_shared:
                    pltpu.sync_copy(u_hbm.at[pl.ds(base, CH)], uv)
                compute_chunk(base, binsv[0], wv[0], outv[0])
                pltpu.sync_copy(outv[0], out_hbm.at[pl.ds(base, CH)])
                return 0

            lax.fori_loop(0, stages, stage_body, 0)
            return

        def start_in(b, base):
            pltpu.async_copy(bins_hbm.at[pl.ds(base, CH)], binsv[b], in_sems[b])
            pltpu.async_copy(w_hbm.at[pl.ds(base, CH)], wv[b], in_sems[b])

        def wait_in(b, base):
            pltpu.make_async_copy(
                bins_hbm.at[pl.ds(base, CH)], binsv[b], in_sems[b]).wait()
            pltpu.make_async_copy(
                w_hbm.at[pl.ds(base, CH)], wv[b], in_sems[b]).wait()

        def wait_out(b, base):
            pltpu.make_async_copy(
                outv[b], out_hbm.at[pl.ds(base, CH)], out_sems[b]).wait()

        npairs = stages // 2
        start_in(0, wbase)

        def pair_body(i, _):
            s0 = 2 * i
            base0 = wbase + s0 * CH
            base1 = base0 + CH
            start_in(1, base1)
            wait_in(0, base0)

            @pl.when(i > 0)
            def _():
                wait_out(0, base0 - 2 * CH)

            compute_chunk(base0, binsv[0], wv[0], outv[0])
            pltpu.async_copy(outv[0], out_hbm.at[pl.ds(base0, CH)], out_sems[0])

            @pl.when(i < npairs - 1)
            def _():
                start_in(0, base0 + 2 * CH)

            wait_in(1, base1)

            @pl.when(i > 0)
            def _():
                wait_out(1, base1 - 2 * CH)

            compute_chunk(base1, binsv[1], wv[1], outv[1])
            pltpu.async_copy(outv[1], out_hbm.at[pl.ds(base1, CH)], out_sems[1])
            return 0

        lax.fori_loop(0, npairs, pair_body, 0)
        wait_out(0, wbase + (stages - 2) * CH)
        wait_out(1, wbase + (stages - 1) * CH)

    return run(bins, weights, u, gridp)


def kernel(bins, weights, n_samples, det):
    N, _ = bins.shape
    S = bins.shape[-1]

    def det_path():
        lo = jnp.float32(0.5) / n_samples
        hi = jnp.float32(1.0) - jnp.float32(0.5) / n_samples
        u = jnp.linspace(lo, hi, S, dtype=jnp.float32)
        step = (hi - lo) / jnp.float32(max(S - 1, 1))
        step = jnp.where(step > 0, step, jnp.float32(1e-30))
        gridp = jnp.concatenate([
            jnp.full((_L,), step, jnp.float32),
            jnp.full((_L,), lo / step, jnp.float32),
        ])
        return _sample_pdf_sc(bins, weights, u, True, gridp)

    def rand_path():
        u = jax.random.uniform(jax.random.key(1), (N, S), dtype=jnp.float32)
        return _sample_pdf_sc(bins, weights, u, False)

    return lax.cond(jnp.asarray(det) != 0, det_path, rand_path)
